# baseline (speedup 1.0000x reference)
import functools

import jax
import jax.numpy as jnp
from jax import lax
from jax.experimental import pallas as pl
from jax.experimental.pallas import tpu as pltpu
from jax.experimental.pallas import tpu_sc as plsc

NUM_HEADS = 16
SEQ = 512
NPOS = SEQ * SEQ
NTAB = 3375
NTAB_PAD = 3456
NC, NS = 2, 16
NW = NC * NS
HSPLIT = 2
HW = NUM_HEADS // HSPLIT
ROWS_W = SEQ // (NW // HSPLIT)
CROWS = 8
CHUNK = CROWS * SEQ
NCHUNK = ROWS_W // CROWS
LANES = 16

HB = 8


def _mlp_body(coords_t_ref, w1t_ref, b1_ref, w2t_ref, out_ref):
    hid = jnp.dot(w1t_ref[...], coords_t_ref[...],
                  preferred_element_type=jnp.float32) + b1_ref[...]
    hid = jnp.maximum(hid, 0.0)
    logits = jnp.dot(w2t_ref[...], hid, preferred_element_type=jnp.float32)
    out_ref[...] = 16.0 / (1.0 + jnp.exp(-logits))


def _add_body(x_ref, b_ref, o_ref):
    o_ref[...] = x_ref[...] + b_ref[...][None]


def _sc_gather_t(table_t, idx_flat, nheads, hbase):
    mesh = plsc.VectorSubcoreMesh(core_axis_name="c", subcore_axis_name="s")
    hw = nheads // HSPLIT

    @functools.partial(
        pl.kernel,
        out_type=jax.ShapeDtypeStruct((nheads, SEQ, SEQ), jnp.float32),
        mesh=mesh,
        compiler_params=pltpu.CompilerParams(needs_layout_passes=False),
        scratch_types=[
            pltpu.VMEM((hw * NTAB_PAD,), jnp.float32),
            pltpu.VMEM((ROWS_W, SEQ), jnp.int32),
            pltpu.VMEM((hw, CROWS, SEQ), jnp.float32),
            pltpu.VMEM((hw, CROWS, SEQ), jnp.float32),
            pltpu.SemaphoreType.DMA,
            pltpu.SemaphoreType.DMA,
        ],
    )
    def k(tbl_hbm, idx_hbm, out_hbm, tbl_v, idx_v, out_v0, out_v1, sem0, sem1):
        wid = lax.axis_index("s") * NC + lax.axis_index("c")
        h0 = (wid // (NW // HSPLIT)) * hw
        row0 = (wid % (NW // HSPLIT)) * ROWS_W
        pltpu.sync_copy(
            tbl_hbm.at[pl.ds((hbase + h0) * NTAB_PAD, hw * NTAB_PAD)], tbl_v)
        pltpu.sync_copy(idx_hbm.at[pl.ds(row0, ROWS_W), :], idx_v)
        bufs = (out_v0, out_v1)
        sems = (sem0, sem1)
        copies = [None, None]
        for c in range(NCHUNK):
            out_v = bufs[c % 2]
            if copies[c % 2] is not None:
                copies[c % 2].wait()

            @plsc.parallel_loop(0, CHUNK, LANES, unroll=4)
            def body(p, out_v=out_v, off=c * CHUNK):
                q = off + p
                g = idx_v[q // SEQ, pl.ds(q % SEQ, LANES)]
                for h in range(hw):
                    out_v[h, p // SEQ, pl.ds(p % SEQ, LANES)] = plsc.load_gather(
                        tbl_v, [g + h * NTAB_PAD])

            copies[c % 2] = pltpu.async_copy(
                out_v,
                out_hbm.at[pl.ds(h0, hw), pl.ds(row0 + c * CROWS, CROWS), :],
                sems[c % 2])
        for cp in copies:
            if cp is not None:
                cp.wait()

    return k(table_t, idx_flat)


def kernel(input_tensor, W1, b1, W2, coords_table, rel_pos_index, local_window_size):
    coords_t = coords_table.reshape(-1, 3).astype(jnp.float32).T
    coords_tp = jnp.pad(coords_t, ((0, 5), (0, NTAB_PAD - NTAB)))
    w1t_p = jnp.pad(W1.astype(jnp.float32).T, ((0, 0), (0, 5)))

    table_t = pl.pallas_call(
        _mlp_body,
        out_shape=jax.ShapeDtypeStruct((NUM_HEADS, NTAB_PAD), jnp.float32),
    )(coords_tp, w1t_p, b1.reshape(-1, 1).astype(jnp.float32),
      W2.astype(jnp.float32).T)

    idx = rel_pos_index.astype(jnp.int32)
    tbl_flat = table_t.reshape(-1)
    nb, nh = input_tensor.shape[0], input_tensor.shape[1]
    halves = []
    for part in range(2):
        hbase = part * (nh // 2)
        bias_half = _sc_gather_t(tbl_flat, idx, nh // 2, hbase)
        halves.append(pl.pallas_call(
            _add_body,
            grid=(1, nb),
            in_specs=[
                pl.BlockSpec((1, HB, SEQ, SEQ),
                             lambda hb, b, hi=hbase // HB: (b, hi, 0, 0)),
                pl.BlockSpec((HB, SEQ, SEQ), lambda hb, b: (0, 0, 0)),
            ],
            out_specs=pl.BlockSpec((1, HB, SEQ, SEQ),
                                   lambda hb, b: (b, 0, 0, 0)),
            out_shape=jax.ShapeDtypeStruct((nb, nh // 2, SEQ, SEQ),
                                           input_tensor.dtype),
        )(input_tensor, bias_half))
    return jnp.concatenate(halves, axis=1)


# speedup vs baseline: 1.7757x; 1.7757x over previous
import functools

import jax
import jax.numpy as jnp
from jax import lax
from jax.experimental import pallas as pl
from jax.experimental.pallas import tpu as pltpu
from jax.experimental.pallas import tpu_sc as plsc

NUM_HEADS = 16
SEQ = 512
NPOS = SEQ * SEQ
NTAB = 3375
NTAB_PAD = 3456
NC, NS = 2, 16
NW = NC * NS
HSPLIT = 2
HW = NUM_HEADS // HSPLIT
ROWS_W = SEQ // (NW // HSPLIT)
CROWS = 8
CHUNK = CROWS * SEQ
NCHUNK = ROWS_W // CROWS
LANES = 16

HB = 8


def _mlp_body(coords_t_ref, w1t_ref, b1_ref, w2t_ref, out_ref):
    hid = jnp.dot(w1t_ref[...], coords_t_ref[...],
                  preferred_element_type=jnp.float32) + b1_ref[...]
    hid = jnp.maximum(hid, 0.0)
    logits = jnp.dot(w2t_ref[...], hid, preferred_element_type=jnp.float32)
    out_ref[...] = 16.0 / (1.0 + jnp.exp(-logits))


def _add_body(x_ref, b_ref, o_ref):
    o_ref[...] = x_ref[...] + b_ref[...][None]


def _sc_gather_t(table_t, idx_flat):
    mesh = plsc.VectorSubcoreMesh(core_axis_name="c", subcore_axis_name="s")

    @functools.partial(
        pl.kernel,
        out_type=jax.ShapeDtypeStruct((NUM_HEADS, SEQ, SEQ), jnp.float32),
        mesh=mesh,
        compiler_params=pltpu.CompilerParams(needs_layout_passes=False),
        scratch_types=[
            pltpu.VMEM((HW * NTAB_PAD,), jnp.float32),
            pltpu.VMEM((ROWS_W, SEQ), jnp.int32),
            pltpu.VMEM((HW, CROWS, SEQ), jnp.float32),
            pltpu.VMEM((HW, CROWS, SEQ), jnp.float32),
            pltpu.SemaphoreType.DMA,
            pltpu.SemaphoreType.DMA,
        ],
    )
    def k(tbl_hbm, idx_hbm, out_hbm, tbl_v, idx_v, out_v0, out_v1, sem0, sem1):
        wid = lax.axis_index("s") * NC + lax.axis_index("c")
        h0 = (wid // (NW // HSPLIT)) * HW
        row0 = (wid % (NW // HSPLIT)) * ROWS_W
        pltpu.sync_copy(tbl_hbm.at[pl.ds(h0 * NTAB_PAD, HW * NTAB_PAD)], tbl_v)
        pltpu.sync_copy(idx_hbm.at[pl.ds(row0, ROWS_W), :], idx_v)
        bufs = (out_v0, out_v1)
        sems = (sem0, sem1)
        copies = [None, None]
        for c in range(NCHUNK):
            out_v = bufs[c % 2]
            if copies[c % 2] is not None:
                copies[c % 2].wait()

            @plsc.parallel_loop(0, CHUNK, LANES, unroll=8)
            def body(p, out_v=out_v, off=c * CHUNK):
                q = off + p
                g = idx_v[q // SEQ, pl.ds(q % SEQ, LANES)]
                for h in range(HW):
                    out_v[h, p // SEQ, pl.ds(p % SEQ, LANES)] = plsc.load_gather(
                        tbl_v, [g + h * NTAB_PAD])

            copies[c % 2] = pltpu.async_copy(
                out_v,
                out_hbm.at[pl.ds(h0, HW), pl.ds(row0 + c * CROWS, CROWS), :],
                sems[c % 2])
        for cp in copies:
            if cp is not None:
                cp.wait()

    return k(table_t, idx_flat)


def kernel(input_tensor, W1, b1, W2, coords_table, rel_pos_index, local_window_size):
    coords_t = coords_table.reshape(-1, 3).astype(jnp.float32).T
    coords_tp = jnp.pad(coords_t, ((0, 5), (0, NTAB_PAD - NTAB)))
    w1t_p = jnp.pad(W1.astype(jnp.float32).T, ((0, 0), (0, 5)))

    table_t = pl.pallas_call(
        _mlp_body,
        out_shape=jax.ShapeDtypeStruct((NUM_HEADS, NTAB_PAD), jnp.float32),
    )(coords_tp, w1t_p, b1.reshape(-1, 1).astype(jnp.float32),
      W2.astype(jnp.float32).T)

    idx = rel_pos_index.astype(jnp.int32)
    bias3 = _sc_gather_t(table_t.reshape(-1), idx)

    nb, nh = input_tensor.shape[0], input_tensor.shape[1]
    out = pl.pallas_call(
        _add_body,
        grid=(nh // HB, nb),
        in_specs=[
            pl.BlockSpec((1, HB, SEQ, SEQ), lambda hb, b: (b, hb, 0, 0)),
            pl.BlockSpec((HB, SEQ, SEQ), lambda hb, b: (hb, 0, 0)),
        ],
        out_specs=pl.BlockSpec((1, HB, SEQ, SEQ), lambda hb, b: (b, hb, 0, 0)),
        out_shape=jax.ShapeDtypeStruct(input_tensor.shape, input_tensor.dtype),
    )(input_tensor, bias3)
    return out
